# baseline (speedup 1.0000x reference)
import jax
import jax.numpy as jnp
from jax.experimental import pallas as pl

NUM_EXPERTS = 64


def _body(mask_ref, routes_ref, imp_ref, load_ref):
    i = pl.program_id(0)
    rows = mask_ref.shape[0]
    col = jax.lax.broadcasted_iota(jnp.int32, (rows, NUM_EXPERTS), 1)
    blk = (col == i).astype(mask_ref.dtype)
    mask_ref[...] = blk
    routes_ref[...] = blk
    s = jnp.sum(blk, axis=0)

    @pl.when(i == 0)
    def _():
        imp_ref[...] = jnp.zeros_like(imp_ref)
        load_ref[...] = jnp.zeros_like(load_ref)

    imp_ref[...] += s
    load_ref[...] += s


def kernel(x):
    n = x.shape[0]
    assert n % NUM_EXPERTS == 0, "contiguous equal-block routing"
    rows = n // NUM_EXPERTS
    dt = x.dtype
    out_shape = (
        jax.ShapeDtypeStruct((n, NUM_EXPERTS), dt),
        jax.ShapeDtypeStruct((n, NUM_EXPERTS), dt),
        jax.ShapeDtypeStruct((NUM_EXPERTS,), dt),
        jax.ShapeDtypeStruct((NUM_EXPERTS,), dt),
    )
    mask, routes, imp, load = pl.pallas_call(
        _body,
        grid=(NUM_EXPERTS,),
        out_specs=(
            pl.BlockSpec((rows, NUM_EXPERTS), lambda i: (i, 0)),
            pl.BlockSpec((rows, NUM_EXPERTS), lambda i: (i, 0)),
            pl.BlockSpec((NUM_EXPERTS,), lambda i: (0,)),
            pl.BlockSpec((NUM_EXPERTS,), lambda i: (0,)),
        ),
        out_shape=out_shape,
    )()
    return (mask, routes, imp, load)
